# baseline (speedup 1.0000x reference)
import functools

import jax
import jax.numpy as jnp
from jax import lax
from jax.experimental import pallas as pl
from jax.experimental.pallas import tpu as pltpu
from jax.experimental.pallas import tpu_sc as plsc

_NUM_Q = 100000
_D = 200
_B = 4096
_L = 50
_LP = 56
_DB = _D - 128

_info = plsc.get_sparse_core_info()
_NC = _info.num_cores
_NS = _info.num_subcores
_NW = _NC * _NS
_ROWS_W = _B // _NW
_NBUF = 4
_GROUPS = _ROWS_W // _NBUF

_mesh = plsc.VectorSubcoreMesh(core_axis_name="c", subcore_axis_name="s")


@functools.partial(
    pl.kernel,
    out_type=jax.ShapeDtypeStruct((_B, _L, _D), jnp.float32),
    mesh=_mesh,
    scratch_types=[
        pltpu.VMEM((_ROWS_W * _LP,), jnp.int32),
        pltpu.VMEM((_LP, 128), jnp.float32),
        pltpu.VMEM((_LP, 128), jnp.float32),
        pltpu.VMEM((_LP, 128), jnp.float32),
        pltpu.VMEM((_LP, 128), jnp.float32),
        pltpu.VMEM((_LP, 128), jnp.float32),
        pltpu.VMEM((_LP, 128), jnp.float32),
        pltpu.VMEM((_LP, 128), jnp.float32),
        pltpu.VMEM((_LP, 128), jnp.float32),
        pltpu.VMEM((_LP, _DB), jnp.float32),
        pltpu.VMEM((_LP, _DB), jnp.float32),
        pltpu.VMEM((_LP, _DB), jnp.float32),
        pltpu.VMEM((_LP, _DB), jnp.float32),
        pltpu.SemaphoreType.DMA,
        pltpu.SemaphoreType.DMA,
        pltpu.SemaphoreType.DMA,
        pltpu.SemaphoreType.DMA,
        pltpu.SemaphoreType.DMA,
        pltpu.SemaphoreType.DMA,
        pltpu.SemaphoreType.DMA,
        pltpu.SemaphoreType.DMA,
    ],
)
def _gather(table_hbm, tblb_hbm, idx_hbm, out_hbm, idx_v,
            a0, a1, a2, a3, b0, b1, b2, b3, c0, c1, c2, c3,
            g0, g1, g2, g3, s0, s1, s2, s3):
    bufa = (a0, a1, a2, a3)
    bufb = (b0, b1, b2, b3)
    bufc = (c0, c1, c2, c3)
    gsem = (g0, g1, g2, g3)
    ssem = (s0, s1, s2, s3)
    wid = lax.axis_index("s") * _NC + lax.axis_index("c")
    base = wid * _ROWS_W
    pltpu.sync_copy(idx_hbm.at[pl.ds(wid * _ROWS_W * _LP, _ROWS_W * _LP)],
                    idx_v)

    def start_gathers(j, b):
        isl = idx_v.at[pl.ds(j * _LP, _LP)]
        pltpu.async_copy(table_hbm.at[isl, pl.ds(0, 128)], bufa[b], gsem[b])
        pltpu.async_copy(tblb_hbm.at[isl], bufb[b], gsem[b])

    def wait_gathers(b):
        pltpu.make_async_copy(table_hbm.at[pl.ds(0, _LP), pl.ds(0, 128)],
                              bufa[b], gsem[b]).wait()
        pltpu.make_async_copy(tblb_hbm.at[pl.ds(0, _LP)],
                              bufb[b], gsem[b]).wait()

    def bridge(b):
        def brow(j, carry):
            for k in (0, 16, 32, 48, _DB - 16):
                bufc[b][j, pl.ds(k, 16)] = bufb[b][j, pl.ds(k, 16)]
            return carry

        lax.fori_loop(0, _L, brow, 0)

    def start_stores(j, b):
        row = base + j
        pltpu.async_copy(bufa[b].at[pl.ds(0, _L)],
                         out_hbm.at[row, :, pl.ds(0, 128)], ssem[b])
        pltpu.async_copy(bufc[b].at[pl.ds(0, _L)],
                         out_hbm.at[row, :, pl.ds(128, _DB)], ssem[b])

    def wait_stores(b):
        pltpu.make_async_copy(bufa[b].at[pl.ds(0, _L)],
                              out_hbm.at[0, :, pl.ds(0, 128)], ssem[b]).wait()
        pltpu.make_async_copy(bufc[b].at[pl.ds(0, _L)],
                              out_hbm.at[0, :, pl.ds(128, _DB)],
                              ssem[b]).wait()

    for b in range(_NBUF):
        start_gathers(b, b)

    def group(g, carry):
        j0 = g * _NBUF
        for b in range(_NBUF):
            wait_gathers(b)
            bridge(b)
            start_stores(j0 + b, b)

            @pl.when(g + 1 < _GROUPS)
            def _():
                wait_stores(b)
                start_gathers(j0 + b + _NBUF, b)
        return carry

    lax.fori_loop(0, _GROUPS, group, 0)
    for b in range(_NBUF):
        wait_stores(b)


def kernel(indices, table):
    tblb = jnp.pad(table[:, 128:], ((0, 0), (0, 128 - _DB)))
    idxp = jnp.pad(indices, ((0, 0), (0, _LP - _L))).reshape(-1)
    return _gather(table, tblb, idxp)
